# baseline (speedup 1.0000x reference)
import jax
import jax.numpy as jnp
from jax import lax
from jax.experimental import pallas as pl
from jax.experimental.pallas import tpu as pltpu
from jax.experimental.pallas import tpu_sc as plsc

N_VOCAB = 1000000
N_EMBED = 64
BATCH = 16384

NC = 2
NS = 16
NW = NC * NS
B_PER_W = BATCH // NW
LANE = 16
ROWS_PER_WIDE = 8
WIDE = ROWS_PER_WIDE * N_EMBED
N_WIDE = N_VOCAB // ROWS_PER_WIDE
CHUNK = 32
N_CHUNKS = B_PER_W // CHUNK


def _gather_body(idx_hbm, table_hbm, out_hbm, idx_v, hi_v, buf0, buf1, outbuf_v,
                 sem0, sem1):
    bufs = (buf0, buf1)
    sems = (sem0, sem1)
    wid = lax.axis_index("s") * NC + lax.axis_index("c")
    base = wid * B_PER_W
    pltpu.sync_copy(idx_hbm.at[pl.ds(base, B_PER_W)], idx_v)
    for k in range(B_PER_W // LANE):
        v = idx_v[pl.ds(k * LANE, LANE)]
        hi_v[pl.ds(k * LANE, LANE)] = lax.shift_right_logical(v, 3)

    def fire(g, nbuf):
        pltpu.async_copy(
            table_hbm.at[hi_v.at[pl.ds(g * CHUNK, CHUNK)]],
            bufs[nbuf], sems[nbuf],
        )

    fire(0, 0)

    def chunk_pair(i, carry):
        for p in range(2):
            g = i * 2 + p
            cur = p
            @pl.when(g + 1 < N_CHUNKS)
            def _():
                fire(g + 1, 1 - cur)
            pltpu.make_async_copy(
                table_hbm.at[pl.ds(0, CHUNK)], bufs[cur], sems[cur]
            ).wait()
            j0 = pl.multiple_of(g * CHUNK, CHUNK)
            buf = bufs[cur]
            for kb in range(CHUNK // LANE):
                rv = lax.bitwise_and(idx_v[pl.ds(j0 + kb * LANE, LANE)], 7)
                for b in range(LANE):
                    row = kb * LANE + b
                    j = j0 + row
                    col = rv[b] * N_EMBED
                    for c in range(N_EMBED // LANE):
                        outbuf_v[j, pl.ds(c * LANE, LANE)] = (
                            buf[row, pl.ds(col + c * LANE, LANE)]
                        )
        return carry

    lax.fori_loop(0, N_CHUNKS // 2, chunk_pair, jnp.int32(0))
    pltpu.sync_copy(outbuf_v, out_hbm.at[pl.ds(base, B_PER_W)])


@jax.jit
def _gather(idx, tablew):
    return pl.kernel(
        _gather_body,
        mesh=plsc.VectorSubcoreMesh(core_axis_name="c", subcore_axis_name="s"),
        out_type=jax.ShapeDtypeStruct((BATCH, N_EMBED), jnp.float32),
        scratch_types=[
            pltpu.VMEM((B_PER_W,), jnp.int32),
            pltpu.VMEM((B_PER_W,), jnp.int32),
            pltpu.VMEM((CHUNK, WIDE), jnp.float32),
            pltpu.VMEM((CHUNK, WIDE), jnp.float32),
            pltpu.VMEM((B_PER_W, N_EMBED), jnp.float32),
            pltpu.SemaphoreType.DMA,
            pltpu.SemaphoreType.DMA,
        ],
    )(idx, tablew)


def kernel(input_words, W_in):
    idx = input_words.astype(jnp.int32)
    tablew = W_in.reshape(N_WIDE, WIDE)
    return _gather(idx, tablew)


# speedup vs baseline: 1.0359x; 1.0359x over previous
import jax
import jax.numpy as jnp
from jax import lax
from jax.experimental import pallas as pl
from jax.experimental.pallas import tpu as pltpu
from jax.experimental.pallas import tpu_sc as plsc

N_VOCAB = 1000000
N_EMBED = 64
BATCH = 16384

NC = 2
NS = 16
NW = NC * NS
B_PER_W = BATCH // NW
CHUNK = 128
N_CHUNKS = B_PER_W // CHUNK


def _gather_body(idx_hbm, table_hbm, out_hbm, idx_v, rows_v, sem):
    wid = lax.axis_index("s") * NC + lax.axis_index("c")
    base = wid * B_PER_W
    pltpu.sync_copy(idx_hbm.at[pl.ds(base, B_PER_W)], idx_v)
    copies = [
        pltpu.async_copy(
            table_hbm.at[idx_v.at[pl.ds(j * CHUNK, CHUNK)]],
            rows_v.at[pl.ds(j * CHUNK, CHUNK)],
            sem,
        )
        for j in range(N_CHUNKS)
    ]
    for c in copies:
        c.wait()
    pltpu.sync_copy(rows_v, out_hbm.at[pl.ds(base, B_PER_W)])


@jax.jit
def _gather(idx, table):
    return pl.kernel(
        _gather_body,
        mesh=plsc.VectorSubcoreMesh(core_axis_name="c", subcore_axis_name="s"),
        out_type=jax.ShapeDtypeStruct((BATCH, N_EMBED), jnp.float32),
        scratch_types=[
            pltpu.VMEM((B_PER_W,), jnp.int32),
            pltpu.VMEM((B_PER_W, N_EMBED), jnp.float32),
            pltpu.SemaphoreType.DMA,
        ],
        compiler_params=pltpu.CompilerParams(use_tc_tiling_on_sc=False),
    )(idx, table)


def kernel(input_words, W_in):
    return _gather(input_words.astype(jnp.int32), W_in)


# speedup vs baseline: 2.6283x; 2.5371x over previous
import jax
import jax.numpy as jnp
from jax import lax
from jax.experimental import pallas as pl
from jax.experimental.pallas import tpu as pltpu
from jax.experimental.pallas import tpu_sc as plsc

N_VOCAB = 1000000
N_EMBED = 64
BATCH = 16384

NC = 2
NS = 16
NW = NC * NS
B_PER_W = BATCH // NW
LANE = 16
ROWS_PER_TILE = 8
N_TILE_ROWS = N_VOCAB // ROWS_PER_TILE
WAVE = 64
N_WAVES = B_PER_W // WAVE


def _gather_body(idx_hbm, table_hbm, out_hbm, idx_v, outbuf_v, sem):
    wid = lax.axis_index("s") * NC + lax.axis_index("c")
    base = wid * B_PER_W
    pltpu.sync_copy(idx_hbm.at[pl.ds(base, B_PER_W)], idx_v)

    def fire_wave(w):
        for kb in range(WAVE // LANE):
            j0 = pl.multiple_of(w * WAVE + kb * LANE, LANE)
            v = idx_v[pl.ds(j0, LANE)]
            hv = lax.shift_right_logical(v, 3)
            rv = lax.bitwise_and(v, 7)
            for b in range(LANE):
                pltpu.async_copy(
                    table_hbm.at[hv[b], rv[b]], outbuf_v.at[j0 + b], sem
                )

    def wait_wave(w):
        pltpu.make_async_copy(
            out_hbm.at[pl.ds(0, WAVE)],
            outbuf_v.at[pl.ds(w * WAVE, WAVE)],
            sem,
        ).wait()

    fire_wave(0)

    def wave_body(i, carry):
        @pl.when(i + 1 < N_WAVES)
        def _():
            fire_wave_dyn(i + 1)
        wait_wave_dyn(i)
        return carry

    def fire_wave_dyn(w):
        for kb in range(WAVE // LANE):
            j0 = pl.multiple_of(w * WAVE + kb * LANE, LANE)
            v = idx_v[pl.ds(j0, LANE)]
            hv = lax.shift_right_logical(v, 3)
            rv = lax.bitwise_and(v, 7)
            for b in range(LANE):
                pltpu.async_copy(
                    table_hbm.at[hv[b], rv[b]], outbuf_v.at[j0 + b], sem
                )

    def wait_wave_dyn(w):
        pltpu.make_async_copy(
            out_hbm.at[pl.ds(0, WAVE)],
            outbuf_v.at[pl.ds(w * WAVE, WAVE)],
            sem,
        ).wait()

    lax.fori_loop(0, N_WAVES, wave_body, jnp.int32(0))
    pltpu.sync_copy(outbuf_v, out_hbm.at[pl.ds(base, B_PER_W)])


@jax.jit
def _gather(idx, table3):
    return pl.kernel(
        _gather_body,
        mesh=plsc.VectorSubcoreMesh(core_axis_name="c", subcore_axis_name="s"),
        out_type=jax.ShapeDtypeStruct((BATCH, N_EMBED), jnp.float32),
        scratch_types=[
            pltpu.VMEM((B_PER_W,), jnp.int32),
            pltpu.VMEM((B_PER_W, N_EMBED), jnp.float32),
            pltpu.SemaphoreType.DMA,
        ],
    )(idx, table3)


def kernel(input_words, W_in):
    idx = input_words.astype(jnp.int32)
    table3 = W_in.reshape(N_TILE_ROWS, ROWS_PER_TILE, N_EMBED)
    return _gather(idx, table3)
